# baseline (speedup 1.0000x reference)
import functools

import jax
import jax.numpy as jnp
from jax import lax
from jax.experimental import pallas as pl
from jax.experimental.pallas import tpu as pltpu
from jax.experimental.pallas import tpu_sc as plsc

_NSEG = 4096
_NIN = 128
_NHID = 128
_BLK = 2000
_C = 512
_NSUB = 16


def _silu(x):
    return x * jax.nn.sigmoid(x)


def _tc_body(s_ref, v_ref, pos_ref, w1v_ref, w1as_ref, w1an_ref, b1a_ref,
             w1b_ref, b1b_ref, w2v3_ref, w2as_ref, w2an_ref, b2a_ref,
             w2b_ref, b2b_ref, o_ref):
    f32 = jnp.float32
    nh = _NHID
    blk = s_ref.shape[0]
    s = s_ref[...]
    w1v = w1v_ref[...]

    vm0 = jnp.dot(v_ref[0], w1v, preferred_element_type=f32)
    vm1 = jnp.dot(v_ref[1], w1v, preferred_element_type=f32)
    vm2 = jnp.dot(v_ref[2], w1v, preferred_element_type=f32)

    vn = jnp.sqrt(vm0[:, :nh] ** 2 + vm1[:, :nh] ** 2 + vm2[:, :nh] ** 2)
    h = _silu(jnp.dot(s, w1as_ref[...], preferred_element_type=f32)
              + jnp.dot(vn, w1an_ref[...], preferred_element_type=f32)
              + b1a_ref[...])
    x = jnp.dot(h, w1b_ref[...], preferred_element_type=f32) + b1b_ref[...]
    s1 = _silu(x[:, :nh])
    g1 = x[:, nh:]
    l1cat = jnp.concatenate(
        [g1 * vm0[:, nh:], g1 * vm1[:, nh:], g1 * vm2[:, nh:]], axis=1)

    ab = jnp.dot(l1cat, w2v3_ref[...], preferred_element_type=f32)
    a0 = ab[:, 0:1]
    a1 = ab[:, 1:2]
    a2 = ab[:, 2:3]
    b3 = ab[:, 3:6]
    vn2 = jnp.sqrt(a0 * a0 + a1 * a1 + a2 * a2)

    h2 = _silu(jnp.dot(s1, w2as_ref[...], preferred_element_type=f32)
               + vn2 * w2an_ref[...]
               + b2a_ref[...])
    qg = jnp.dot(h2, w2b_ref[...], preferred_element_type=f32) + b2b_ref[...]
    q = qg[:, 0:1]
    g2 = qg[:, 1:2]

    d3 = g2 * b3
    y3 = d3 + pos_ref[...] * q

    o_ref[...] = jnp.concatenate(
        [y3, d3, jnp.zeros((blk, 10), f32)], axis=1)


def _atom_mlp(s, v2d, pos, w1v, w1as, w1an, b1a2, w1b, b1b2, w2vt, w2as,
              w2an2, b2a2, w2bt, b2b2):
    n = s.shape[0]
    blk = _BLK
    grid = pl.cdiv(n, blk)
    row = lambda d: pl.BlockSpec((blk, d), lambda i: (i, 0))
    full = lambda d0, d1: pl.BlockSpec((d0, d1), lambda i: (0, 0))
    return pl.pallas_call(
        _tc_body,
        grid=(grid,),
        in_specs=[
            row(_NIN),
            pl.BlockSpec((3, blk, _NIN), lambda i: (0, i, 0)),
            row(3),
            full(_NIN, 2 * _NHID),
            full(_NIN, _NHID),
            full(_NHID, _NHID),
            full(1, _NHID),
            full(_NHID, 2 * _NHID),
            full(1, 2 * _NHID),
            full(3 * _NHID, 6),
            full(_NHID, _NHID),
            full(1, _NHID),
            full(1, _NHID),
            full(_NHID, 2),
            full(1, 2),
        ],
        out_specs=row(16),
        out_shape=jax.ShapeDtypeStruct((n, 16), jnp.float32),
    )(s, v2d, pos, w1v, w1as, w1an, b1a2, w1b, b1b2, w2vt, w2as, w2an2,
      b2a2, w2bt, b2b2)


def _make_segsum(n):
    nw = 2 * _NSUB
    nfull = n // _C
    tail = n - nfull * _C
    per = pl.cdiv(nfull, nw)
    mesh = plsc.VectorSubcoreMesh(core_axis_name="c", subcore_axis_name="s")

    @functools.partial(
        pl.kernel,
        mesh=mesh,
        out_type=jax.ShapeDtypeStruct((nw, _NSEG * 16), jnp.float32),
        scratch_types=[
            pltpu.VMEM((_C,), jnp.int32),
            pltpu.VMEM((_C * 16,), jnp.float32),
            pltpu.VMEM((_NSEG * 16,), jnp.float32),
        ],
    )
    def segsum(rows_hbm, ids_hbm, slabs, ids_v, rows_v, acc):
        cid = lax.axis_index("c")
        sid = lax.axis_index("s")
        wid = sid * 2 + cid
        zero = jnp.zeros((16,), jnp.float32)

        def zbody(i, cr):
            acc[pl.ds(i * 16, 16)] = zero
            return cr
        lax.fori_loop(0, _NSEG, zbody, 0)

        def accum_rows(nrows):
            def rbody(r, cr):
                bvec = ids_v[pl.ds(r * 16, 16)]
                for l in range(16):
                    off = (r * 16 + l) * 16
                    plsc.addupdate(acc.at[pl.ds(bvec[l] * 16, 16)],
                                   rows_v[pl.ds(off, 16)])
                return cr
            lax.fori_loop(0, nrows // 16, rbody, 0)

        def cbody(j, cr):
            g = wid + nw * j

            @pl.when(g < nfull)
            def _():
                pltpu.sync_copy(ids_hbm.at[pl.ds(g * _C, _C)], ids_v)
                pltpu.sync_copy(rows_hbm.at[pl.ds(g * _C * 16, _C * 16)],
                                rows_v)
                accum_rows(_C)
            return cr
        lax.fori_loop(0, per, cbody, 0)

        if tail:
            @pl.when(wid == nw - 1)
            def _():
                pltpu.sync_copy(ids_hbm.at[pl.ds(nfull * _C, tail)],
                                ids_v.at[pl.ds(0, tail)])
                pltpu.sync_copy(
                    rows_hbm.at[pl.ds(nfull * _C * 16, tail * 16)],
                    rows_v.at[pl.ds(0, tail * 16)])
                accum_rows(tail)

        pltpu.sync_copy(acc, slabs.at[wid])

    return segsum


def _merge_body(slabs_ref, y_ref, yv_ref):
    ssum = jnp.sum(slabs_ref[...], axis=0)
    y_ref[...] = ssum[:, 0:3]
    yv_ref[...] = ssum[:, 3:6]


def _merge(slabs3):
    nw = slabs3.shape[0]
    sb = 512
    grid = _NSEG // sb
    return pl.pallas_call(
        _merge_body,
        grid=(grid,),
        in_specs=[pl.BlockSpec((nw, sb, 16), lambda j: (0, j, 0))],
        out_specs=[pl.BlockSpec((sb, 3), lambda j: (j, 0)),
                   pl.BlockSpec((sb, 3), lambda j: (j, 0))],
        out_shape=[jax.ShapeDtypeStruct((_NSEG, 3), jnp.float32),
                   jax.ShapeDtypeStruct((_NSEG, 3), jnp.float32)],
    )(slabs3)


def kernel(pos, representation, vector_representation, W1v, W1a, b1a, W1b,
           b1b, W2v, W2a, b2a, W2b, b2b, batch):
    n = pos.shape[0]
    nh = _NHID
    v3 = jnp.transpose(vector_representation, (1, 0, 2))

    w2v3 = jnp.zeros((3 * nh, 6), jnp.float32)
    w2v3 = w2v3.at[0 * nh:1 * nh, 0].set(W2v[:, 0])
    w2v3 = w2v3.at[1 * nh:2 * nh, 1].set(W2v[:, 0])
    w2v3 = w2v3.at[2 * nh:3 * nh, 2].set(W2v[:, 0])
    w2v3 = w2v3.at[0 * nh:1 * nh, 3].set(W2v[:, 1])
    w2v3 = w2v3.at[1 * nh:2 * nh, 4].set(W2v[:, 1])
    w2v3 = w2v3.at[2 * nh:3 * nh, 5].set(W2v[:, 1])

    o16 = _atom_mlp(
        representation, v3, pos,
        W1v,
        W1a[:_NIN], W1a[_NIN:],
        b1a.reshape(1, nh),
        W1b, b1b.reshape(1, 2 * nh),
        w2v3,
        W2a[:nh], W2a[nh:nh + 1],
        b2a.reshape(1, nh),
        W2b, b2b.reshape(1, 2),
    )

    slabs = _make_segsum(n)(o16.reshape(-1), batch)
    y, yv = _merge(slabs.reshape(2 * _NSUB, _NSEG, 16))
    return (y, yv[..., None])


# speedup vs baseline: 1.0728x; 1.0728x over previous
import functools

import jax
import jax.numpy as jnp
from jax import lax
from jax.experimental import pallas as pl
from jax.experimental.pallas import tpu as pltpu
from jax.experimental.pallas import tpu_sc as plsc

_NSEG = 4096
_NIN = 128
_NHID = 128
_BLK = 2000
_C = 512
_NSUB = 16


def _silu(x):
    return x * jax.nn.sigmoid(x)


def _tc_body(s_ref, v_ref, pos_ref, w1v_ref, w1as_ref, w1an_ref, b1a_ref,
             w1b_ref, b1b_ref, w2v3_ref, w2as_ref, w2an_ref, b2a_ref,
             w2b_ref, b2b_ref, o_ref):
    f32 = jnp.float32
    bf16 = jnp.bfloat16
    nh = _NHID
    blk = s_ref.shape[0]
    s = s_ref[...]
    w1v = w1v_ref[...].astype(bf16)

    vm0 = jnp.dot(v_ref[0].astype(bf16), w1v, preferred_element_type=f32)
    vm1 = jnp.dot(v_ref[1].astype(bf16), w1v, preferred_element_type=f32)
    vm2 = jnp.dot(v_ref[2].astype(bf16), w1v, preferred_element_type=f32)

    vn = jnp.sqrt(vm0[:, :nh] ** 2 + vm1[:, :nh] ** 2 + vm2[:, :nh] ** 2)
    h = _silu(jnp.dot(s, w1as_ref[...], preferred_element_type=f32)
              + jnp.dot(vn, w1an_ref[...], preferred_element_type=f32)
              + b1a_ref[...])
    x = jnp.dot(h, w1b_ref[...], preferred_element_type=f32) + b1b_ref[...]
    s1 = _silu(x[:, :nh])
    g1 = x[:, nh:]
    l10 = g1 * vm0[:, nh:]
    l11 = g1 * vm1[:, nh:]
    l12 = g1 * vm2[:, nh:]

    w2v3 = w2v3_ref[...]
    ab = (jnp.dot(l10, w2v3[0 * nh:1 * nh], preferred_element_type=f32)
          + jnp.dot(l11, w2v3[1 * nh:2 * nh], preferred_element_type=f32)
          + jnp.dot(l12, w2v3[2 * nh:3 * nh], preferred_element_type=f32))
    a0 = ab[:, 0:1]
    a1 = ab[:, 1:2]
    a2 = ab[:, 2:3]
    b3 = ab[:, 3:6]
    vn2 = jnp.sqrt(a0 * a0 + a1 * a1 + a2 * a2)

    h2 = _silu(jnp.dot(s1, w2as_ref[...], preferred_element_type=f32)
               + vn2 * w2an_ref[...]
               + b2a_ref[...])
    qg = jnp.dot(h2, w2b_ref[...], preferred_element_type=f32) + b2b_ref[...]
    q = qg[:, 0:1]
    g2 = qg[:, 1:2]

    d3 = g2 * b3
    y3 = d3 + pos_ref[...] * q

    o_ref[...] = jnp.concatenate(
        [y3, d3, jnp.zeros((blk, 10), f32)], axis=1)


def _atom_mlp(s, v2d, pos, w1v, w1as, w1an, b1a2, w1b, b1b2, w2vt, w2as,
              w2an2, b2a2, w2bt, b2b2):
    n = s.shape[0]
    blk = _BLK
    grid = pl.cdiv(n, blk)
    row = lambda d: pl.BlockSpec((blk, d), lambda i: (i, 0))
    full = lambda d0, d1: pl.BlockSpec((d0, d1), lambda i: (0, 0))
    return pl.pallas_call(
        _tc_body,
        grid=(grid,),
        in_specs=[
            row(_NIN),
            pl.BlockSpec((3, blk, _NIN), lambda i: (0, i, 0)),
            row(3),
            full(_NIN, 2 * _NHID),
            full(_NIN, _NHID),
            full(_NHID, _NHID),
            full(1, _NHID),
            full(_NHID, 2 * _NHID),
            full(1, 2 * _NHID),
            full(3 * _NHID, 6),
            full(_NHID, _NHID),
            full(1, _NHID),
            full(1, _NHID),
            full(_NHID, 2),
            full(1, 2),
        ],
        out_specs=row(16),
        out_shape=jax.ShapeDtypeStruct((n, 16), jnp.float32),
    )(s, v2d, pos, w1v, w1as, w1an, b1a2, w1b, b1b2, w2vt, w2as, w2an2,
      b2a2, w2bt, b2b2)


def _make_segsum(n):
    nfull = n // _C
    tail = n - nfull * _C
    per = pl.cdiv(nfull, _NSUB)
    seg_per = _NSEG // _NSUB
    mesh = plsc.VectorSubcoreMesh(core_axis_name="c", subcore_axis_name="s")

    @functools.partial(
        pl.kernel,
        mesh=mesh,
        out_type=(
            jax.ShapeDtypeStruct((_NSEG * 16,), jnp.float32),
            jax.ShapeDtypeStruct((_NSUB, _NSEG * 16), jnp.float32),
        ),
        scratch_types=[
            pltpu.VMEM((_C,), jnp.int32),
            pltpu.VMEM((_C * 16,), jnp.float32),
            pltpu.VMEM((_NSEG * 16,), jnp.float32),
            pltpu.VMEM((seg_per * 16,), jnp.float32),
            pltpu.VMEM((seg_per * 16,), jnp.float32),
        ],
    )
    def segsum(rows_hbm, ids_hbm, out, slabs, ids_v, rows_v, acc, mbuf, msum):
        cid = lax.axis_index("c")
        sid = lax.axis_index("s")

        @pl.when(cid == 0)
        def _():
            zero = jnp.zeros((16,), jnp.float32)

            def zbody(i, cr):
                acc[pl.ds(i * 16, 16)] = zero
                return cr
            lax.fori_loop(0, _NSEG, zbody, 0)

            def accum_rows(nrows):
                def rbody(r, cr):
                    bvec = ids_v[pl.ds(r * 16, 16)]
                    for l in range(16):
                        off = (r * 16 + l) * 16
                        plsc.addupdate(acc.at[pl.ds(bvec[l] * 16, 16)],
                                       rows_v[pl.ds(off, 16)])
                    return cr
                lax.fori_loop(0, nrows // 16, rbody, 0)

            def cbody(j, cr):
                g = sid + _NSUB * j

                @pl.when(g < nfull)
                def _():
                    pltpu.sync_copy(ids_hbm.at[pl.ds(g * _C, _C)], ids_v)
                    pltpu.sync_copy(rows_hbm.at[pl.ds(g * _C * 16, _C * 16)],
                                    rows_v)
                    accum_rows(_C)
                return cr
            lax.fori_loop(0, per, cbody, 0)

            if tail:
                @pl.when(sid == _NSUB - 1)
                def _():
                    pltpu.sync_copy(ids_hbm.at[pl.ds(nfull * _C, tail)],
                                    ids_v.at[pl.ds(0, tail)])
                    pltpu.sync_copy(
                        rows_hbm.at[pl.ds(nfull * _C * 16, tail * 16)],
                        rows_v.at[pl.ds(0, tail * 16)])
                    accum_rows(tail)

            pltpu.sync_copy(acc, slabs.at[sid])
            plsc.subcore_barrier()

            base = sid * seg_per * 16
            pltpu.sync_copy(slabs.at[0, pl.ds(base, seg_per * 16)], msum)

            def tbody(t, cr):
                pltpu.sync_copy(slabs.at[t, pl.ds(base, seg_per * 16)], mbuf)

                def abody(r, cr2):
                    plsc.addupdate(msum.at[pl.ds(r * 16, 16)],
                                   mbuf[pl.ds(r * 16, 16)])
                    return cr2
                lax.fori_loop(0, seg_per, abody, 0)
                return cr
            lax.fori_loop(1, _NSUB, tbody, 0)

            pltpu.sync_copy(msum, out.at[pl.ds(base, seg_per * 16)])

    return segsum


def kernel(pos, representation, vector_representation, W1v, W1a, b1a, W1b,
           b1b, W2v, W2a, b2a, W2b, b2b, batch):
    n = pos.shape[0]
    nh = _NHID
    v3 = jnp.transpose(vector_representation, (1, 0, 2))

    w2v3 = jnp.zeros((3 * nh, 6), jnp.float32)
    w2v3 = w2v3.at[0 * nh:1 * nh, 0].set(W2v[:, 0])
    w2v3 = w2v3.at[1 * nh:2 * nh, 1].set(W2v[:, 0])
    w2v3 = w2v3.at[2 * nh:3 * nh, 2].set(W2v[:, 0])
    w2v3 = w2v3.at[0 * nh:1 * nh, 3].set(W2v[:, 1])
    w2v3 = w2v3.at[1 * nh:2 * nh, 4].set(W2v[:, 1])
    w2v3 = w2v3.at[2 * nh:3 * nh, 5].set(W2v[:, 1])

    o16 = _atom_mlp(
        representation, v3, pos,
        W1v,
        W1a[:_NIN], W1a[_NIN:],
        b1a.reshape(1, nh),
        W1b, b1b.reshape(1, 2 * nh),
        w2v3,
        W2a[:nh], W2a[nh:nh + 1],
        b2a.reshape(1, nh),
        W2b, b2b.reshape(1, 2),
    )

    seg_flat, _ = _make_segsum(n)(o16.reshape(-1), batch)
    seg = seg_flat.reshape(_NSEG, 16)
    y = seg[:, :3]
    y_vector = seg[:, 3:6][..., None]
    return (y, y_vector)


# speedup vs baseline: 1.0857x; 1.0120x over previous
import functools

import jax
import jax.numpy as jnp
from jax import lax
from jax.experimental import pallas as pl
from jax.experimental.pallas import tpu as pltpu
from jax.experimental.pallas import tpu_sc as plsc

_NSEG = 4096
_NIN = 128
_NHID = 128
_BLK = 2000
_C = 512
_NSUB = 16


def _silu(x):
    return x * jax.nn.sigmoid(x)


def _tc_body(s_ref, v_ref, pos_ref, w1v_ref, w1as_ref, w1an_ref, b1a_ref,
             w1b_ref, b1b_ref, w2v3_ref, w2as_ref, w2an_ref, b2a_ref,
             w2b_ref, b2b_ref, o_ref):
    f32 = jnp.float32
    nh = _NHID
    blk = s_ref.shape[0]
    s = s_ref[...]
    w1v = w1v_ref[...]

    vm0 = jnp.dot(v_ref[0], w1v, preferred_element_type=f32)
    vm1 = jnp.dot(v_ref[1], w1v, preferred_element_type=f32)
    vm2 = jnp.dot(v_ref[2], w1v, preferred_element_type=f32)

    vn = jnp.sqrt(vm0[:, :nh] ** 2 + vm1[:, :nh] ** 2 + vm2[:, :nh] ** 2)
    h = _silu(jnp.dot(s, w1as_ref[...], preferred_element_type=f32)
              + jnp.dot(vn, w1an_ref[...], preferred_element_type=f32)
              + b1a_ref[...])
    x = jnp.dot(h, w1b_ref[...], preferred_element_type=f32) + b1b_ref[...]
    s1 = _silu(x[:, :nh])
    g1 = x[:, nh:]
    l1cat = jnp.concatenate(
        [g1 * vm0[:, nh:], g1 * vm1[:, nh:], g1 * vm2[:, nh:]], axis=1)

    ab = jnp.dot(l1cat, w2v3_ref[...], preferred_element_type=f32)
    a0 = ab[:, 0:1]
    a1 = ab[:, 1:2]
    a2 = ab[:, 2:3]
    b3 = ab[:, 3:6]
    vn2 = jnp.sqrt(a0 * a0 + a1 * a1 + a2 * a2)

    h2 = _silu(jnp.dot(s1, w2as_ref[...], preferred_element_type=f32)
               + vn2 * w2an_ref[...]
               + b2a_ref[...])
    qg = jnp.dot(h2, w2b_ref[...], preferred_element_type=f32) + b2b_ref[...]
    q = qg[:, 0:1]
    g2 = qg[:, 1:2]

    d3 = g2 * b3
    y3 = d3 + pos_ref[...] * q

    o_ref[...] = jnp.concatenate([y3, d3, jnp.zeros((blk, 10), f32)], axis=1)


def _atom_mlp(s, v3, pos, w1v, w1as, w1an, b1a2, w1b, b1b2, w2v3, w2as,
              w2an2, b2a2, w2b, b2b2, n_half, blk_ofs):
    blk = _BLK
    grid = n_half // blk
    row = lambda d: pl.BlockSpec((blk, d), lambda i: (i + blk_ofs, 0))
    full = lambda d0, d1: pl.BlockSpec((d0, d1), lambda i: (0, 0))
    return pl.pallas_call(
        _tc_body,
        grid=(grid,),
        in_specs=[
            row(_NIN),
            pl.BlockSpec((3, blk, _NIN), lambda i: (0, i + blk_ofs, 0)),
            row(3),
            full(_NIN, 2 * _NHID),
            full(_NIN, _NHID),
            full(_NHID, _NHID),
            full(1, _NHID),
            full(_NHID, 2 * _NHID),
            full(1, 2 * _NHID),
            full(3 * _NHID, 6),
            full(_NHID, _NHID),
            full(1, _NHID),
            full(1, _NHID),
            full(_NHID, 2),
            full(1, 2),
        ],
        out_specs=pl.BlockSpec((blk, 16), lambda i: (i, 0)),
        out_shape=jax.ShapeDtypeStruct((n_half, 16), jnp.float32),
    )(s, v3, pos, w1v, w1as, w1an, b1a2, w1b, b1b2, w2v3, w2as, w2an2,
      b2a2, w2b, b2b2)


def _sc_accum_phase(nfull, tail, ids_ofs, ids_hbm, rows_hbm, ids_v, rows_v,
                    acc, sid):
    per = pl.cdiv(nfull, _NSUB)
    zero = jnp.zeros((16,), jnp.float32)

    def zbody(i, cr):
        acc[pl.ds(i * 16, 16)] = zero
        return cr
    lax.fori_loop(0, _NSEG, zbody, 0)

    def accum_rows(nrows):
        def rbody(r, cr):
            bvec = ids_v[pl.ds(r * 16, 16)]
            for l in range(16):
                off = (r * 16 + l) * 16
                plsc.addupdate(acc.at[pl.ds(bvec[l] * 16, 16)],
                               rows_v[pl.ds(off, 16)])
            return cr
        lax.fori_loop(0, nrows // 16, rbody, 0)

    def cbody(j, cr):
        g = sid + _NSUB * j

        @pl.when(g < nfull)
        def _():
            pltpu.sync_copy(ids_hbm.at[pl.ds(ids_ofs + g * _C, _C)], ids_v)
            pltpu.sync_copy(rows_hbm.at[pl.ds(g * _C * 16, _C * 16)], rows_v)
            accum_rows(_C)
        return cr
    lax.fori_loop(0, per, cbody, 0)

    if tail:
        @pl.when(sid == _NSUB - 1)
        def _():
            pltpu.sync_copy(ids_hbm.at[pl.ds(ids_ofs + nfull * _C, tail)],
                            ids_v.at[pl.ds(0, tail)])
            pltpu.sync_copy(rows_hbm.at[pl.ds(nfull * _C * 16, tail * 16)],
                            rows_v.at[pl.ds(0, tail * 16)])
            accum_rows(tail)


def _make_segsum_a(n_half):
    nfull = n_half // _C
    tail = n_half - nfull * _C
    mesh = plsc.VectorSubcoreMesh(core_axis_name="c", subcore_axis_name="s")

    @functools.partial(
        pl.kernel,
        mesh=mesh,
        out_type=jax.ShapeDtypeStruct((_NSUB, _NSEG * 16), jnp.float32),
        scratch_types=[
            pltpu.VMEM((_C,), jnp.int32),
            pltpu.VMEM((_C * 16,), jnp.float32),
            pltpu.VMEM((_NSEG * 16,), jnp.float32),
        ],
    )
    def segsum_a(rows_hbm, ids_hbm, slabs, ids_v, rows_v, acc):
        cid = lax.axis_index("c")
        sid = lax.axis_index("s")

        @pl.when(cid == 0)
        def _():
            _sc_accum_phase(nfull, tail, 0, ids_hbm, rows_hbm, ids_v, rows_v,
                            acc, sid)
            pltpu.sync_copy(acc, slabs.at[sid])

    return segsum_a


def _make_segsum_b(n_half, ids_ofs):
    nfull = n_half // _C
    tail = n_half - nfull * _C
    seg_per = _NSEG // _NSUB
    mesh = plsc.VectorSubcoreMesh(core_axis_name="c", subcore_axis_name="s")

    @functools.partial(
        pl.kernel,
        mesh=mesh,
        out_type=(
            jax.ShapeDtypeStruct((_NSEG * 16,), jnp.float32),
            jax.ShapeDtypeStruct((_NSUB, _NSEG * 16), jnp.float32),
        ),
        scratch_types=[
            pltpu.VMEM((_C,), jnp.int32),
            pltpu.VMEM((_C * 16,), jnp.float32),
            pltpu.VMEM((_NSEG * 16,), jnp.float32),
            pltpu.VMEM((_NSEG,), jnp.float32),
            pltpu.VMEM((_NSEG,), jnp.float32),
        ],
    )
    def segsum_b(rows_hbm, ids_hbm, slabs_a, out, slabs, ids_v, rows_v, acc,
                 mbuf, msum):
        cid = lax.axis_index("c")
        sid = lax.axis_index("s")

        @pl.when(cid == 0)
        def _():
            _sc_accum_phase(nfull, tail, ids_ofs, ids_hbm, rows_hbm, ids_v,
                            rows_v, acc, sid)
            pltpu.sync_copy(acc, slabs.at[sid])
            plsc.subcore_barrier()

            base = sid * (_NSEG // _NSUB) * 16
            pltpu.sync_copy(slabs.at[0, pl.ds(base, _NSEG)], msum)

            def merge_from(src_hbm, lo):
                def tbody(t, cr):
                    pltpu.sync_copy(src_hbm.at[t, pl.ds(base, _NSEG)], mbuf)

                    def abody(r, cr2):
                        plsc.addupdate(msum.at[pl.ds(r * 16, 16)],
                                       mbuf[pl.ds(r * 16, 16)])
                        return cr2
                    lax.fori_loop(0, _NSEG // 16, abody, 0)
                    return cr
                lax.fori_loop(lo, _NSUB, tbody, 0)

            merge_from(slabs, 1)
            merge_from(slabs_a, 0)

            pltpu.sync_copy(msum, out.at[pl.ds(base, _NSEG)])

    return segsum_b


def kernel(pos, representation, vector_representation, W1v, W1a, b1a, W1b,
           b1b, W2v, W2a, b2a, W2b, b2b, batch):
    n = pos.shape[0]
    nh = _NHID
    n1 = n // 2
    v3 = jnp.transpose(vector_representation, (1, 0, 2))

    w2v3 = jnp.zeros((3 * nh, 6), jnp.float32)
    w2v3 = w2v3.at[0 * nh:1 * nh, 0].set(W2v[:, 0])
    w2v3 = w2v3.at[1 * nh:2 * nh, 1].set(W2v[:, 0])
    w2v3 = w2v3.at[2 * nh:3 * nh, 2].set(W2v[:, 0])
    w2v3 = w2v3.at[0 * nh:1 * nh, 3].set(W2v[:, 1])
    w2v3 = w2v3.at[1 * nh:2 * nh, 4].set(W2v[:, 1])
    w2v3 = w2v3.at[2 * nh:3 * nh, 5].set(W2v[:, 1])

    weights = (W1v, W1a[:_NIN], W1a[_NIN:], b1a.reshape(1, nh),
               W1b, b1b.reshape(1, 2 * nh), w2v3,
               W2a[:nh], W2a[nh:nh + 1], b2a.reshape(1, nh),
               W2b, b2b.reshape(1, 2))

    o1 = _atom_mlp(representation, v3, pos, *weights, n1, 0)
    o2 = _atom_mlp(representation, v3, pos, *weights, n1, n1 // _BLK)

    slabs_a = _make_segsum_a(n1)(o1.reshape(-1), batch)
    seg_flat, _ = _make_segsum_b(n1, n1)(o2.reshape(-1), batch, slabs_a)
    seg = seg_flat.reshape(_NSEG, 16)
    y = seg[:, :3]
    y_vector = seg[:, 3:6][..., None]
    return (y, y_vector)


# speedup vs baseline: 1.1549x; 1.0637x over previous
import functools

import jax
import jax.numpy as jnp
from jax import lax
from jax.experimental import pallas as pl
from jax.experimental.pallas import tpu as pltpu
from jax.experimental.pallas import tpu_sc as plsc

_NSEG = 4096
_NIN = 128
_NHID = 128
_BLK = 2000
_C = 512
_NSUB = 16


def _silu(x):
    return x * jax.nn.sigmoid(x)


def _tc_body(s_ref, v_ref, pos_ref, w1v_ref, w1as_ref, w1an_ref, b1a_ref,
             w1b_ref, b1b_ref, w2v3_ref, w2as_ref, w2an_ref, b2a_ref,
             w2b_ref, b2b_ref, o_ref):
    f32 = jnp.float32
    nh = _NHID
    blk = s_ref.shape[0]
    s = s_ref[...]
    w1v = w1v_ref[...]

    vm0 = jnp.dot(v_ref[0], w1v, preferred_element_type=f32)
    vm1 = jnp.dot(v_ref[1], w1v, preferred_element_type=f32)
    vm2 = jnp.dot(v_ref[2], w1v, preferred_element_type=f32)

    vn = jnp.sqrt(vm0[:, :nh] ** 2 + vm1[:, :nh] ** 2 + vm2[:, :nh] ** 2)
    h = _silu(jnp.dot(s, w1as_ref[...], preferred_element_type=f32)
              + jnp.dot(vn, w1an_ref[...], preferred_element_type=f32)
              + b1a_ref[...])
    x = jnp.dot(h, w1b_ref[...], preferred_element_type=f32) + b1b_ref[...]
    s1 = _silu(x[:, :nh])
    g1 = x[:, nh:]
    l1cat = jnp.concatenate(
        [g1 * vm0[:, nh:], g1 * vm1[:, nh:], g1 * vm2[:, nh:]], axis=1)

    ab = jnp.dot(l1cat, w2v3_ref[...], preferred_element_type=f32)
    a0 = ab[:, 0:1]
    a1 = ab[:, 1:2]
    a2 = ab[:, 2:3]
    b3 = ab[:, 3:6]
    vn2 = jnp.sqrt(a0 * a0 + a1 * a1 + a2 * a2)

    h2 = _silu(jnp.dot(s1, w2as_ref[...], preferred_element_type=f32)
               + vn2 * w2an_ref[...]
               + b2a_ref[...])
    qg = jnp.dot(h2, w2b_ref[...], preferred_element_type=f32) + b2b_ref[...]
    q = qg[:, 0:1]
    g2 = qg[:, 1:2]

    d3 = g2 * b3
    y3 = d3 + pos_ref[...] * q

    o_ref[...] = jnp.concatenate([y3, d3, jnp.zeros((blk, 10), f32)], axis=1)


def _atom_mlp(s, v3, pos, w1v, w1as, w1an, b1a2, w1b, b1b2, w2v3, w2as,
              w2an2, b2a2, w2b, b2b2, n_half, blk_ofs):
    blk = _BLK
    grid = n_half // blk
    row = lambda d: pl.BlockSpec((blk, d), lambda i: (i + blk_ofs, 0))
    full = lambda d0, d1: pl.BlockSpec((d0, d1), lambda i: (0, 0))
    return pl.pallas_call(
        _tc_body,
        grid=(grid,),
        in_specs=[
            row(_NIN),
            pl.BlockSpec((3, blk, _NIN), lambda i: (0, i + blk_ofs, 0)),
            row(3),
            full(_NIN, 2 * _NHID),
            full(_NIN, _NHID),
            full(_NHID, _NHID),
            full(1, _NHID),
            full(_NHID, 2 * _NHID),
            full(1, 2 * _NHID),
            full(3 * _NHID, 6),
            full(_NHID, _NHID),
            full(1, _NHID),
            full(1, _NHID),
            full(_NHID, 2),
            full(1, 2),
        ],
        out_specs=pl.BlockSpec((blk, 16), lambda i: (i, 0)),
        out_shape=jax.ShapeDtypeStruct((n_half, 16), jnp.float32),
    )(s, v3, pos, w1v, w1as, w1an, b1a2, w1b, b1b2, w2v3, w2as, w2an2,
      b2a2, w2b, b2b2)


def _sc_accum_phase(nfull, tail, ids_ofs, ids_hbm, rows_hbm, ids_v, rows_v,
                    acc, sid):
    per = pl.cdiv(nfull, _NSUB)
    zero = jnp.zeros((16,), jnp.float32)

    def zbody(i, cr):
        acc[pl.ds(i * 16, 16)] = zero
        return cr
    lax.fori_loop(0, _NSEG, zbody, 0)

    def accum_rows(nrows):
        def rbody(r, cr):
            bvec = ids_v[pl.ds(r * 16, 16)]
            for l in range(16):
                off = (r * 16 + l) * 16
                plsc.addupdate(acc.at[pl.ds(bvec[l] * 16, 16)],
                               rows_v[pl.ds(off, 16)])
            return cr
        lax.fori_loop(0, nrows // 16, rbody, 0)

    def cbody(j, cr):
        g = sid + _NSUB * j

        @pl.when(g < nfull)
        def _():
            pltpu.sync_copy(ids_hbm.at[pl.ds(ids_ofs + g * _C, _C)], ids_v)
            pltpu.sync_copy(rows_hbm.at[pl.ds(g * _C * 16, _C * 16)], rows_v)
            accum_rows(_C)
        return cr
    lax.fori_loop(0, per, cbody, 0)

    if tail:
        @pl.when(sid == _NSUB - 1)
        def _():
            pltpu.sync_copy(ids_hbm.at[pl.ds(ids_ofs + nfull * _C, tail)],
                            ids_v.at[pl.ds(0, tail)])
            pltpu.sync_copy(rows_hbm.at[pl.ds(nfull * _C * 16, tail * 16)],
                            rows_v.at[pl.ds(0, tail * 16)])
            accum_rows(tail)


def _make_segsum_a(n_half):
    nfull = n_half // _C
    tail = n_half - nfull * _C
    mesh = plsc.VectorSubcoreMesh(core_axis_name="c", subcore_axis_name="s")

    @functools.partial(
        pl.kernel,
        mesh=mesh,
        out_type=(
            jax.ShapeDtypeStruct((_NSEG * 16,), jnp.float32),
            jax.ShapeDtypeStruct((_NSUB, _NSEG * 16), jnp.float32),
        ),
        scratch_types=[
            pltpu.VMEM((_C,), jnp.int32),
            pltpu.VMEM((_C * 16,), jnp.float32),
            pltpu.VMEM((_NSEG * 16,), jnp.float32),
            pltpu.VMEM((_NSEG,), jnp.float32),
            pltpu.VMEM((_NSEG,), jnp.float32),
        ],
    )
    def segsum_a(rows_hbm, ids_hbm, seg_a, slabs, ids_v, rows_v, acc,
                 mbuf, msum):
        cid = lax.axis_index("c")
        sid = lax.axis_index("s")

        @pl.when(cid == 0)
        def _():
            _sc_accum_phase(nfull, tail, 0, ids_hbm, rows_hbm, ids_v, rows_v,
                            acc, sid)
            pltpu.sync_copy(acc, slabs.at[sid])
            plsc.subcore_barrier()

            base = sid * (_NSEG // _NSUB) * 16
            pltpu.sync_copy(slabs.at[0, pl.ds(base, _NSEG)], msum)

            def tbody(t, cr):
                pltpu.sync_copy(slabs.at[t, pl.ds(base, _NSEG)], mbuf)

                def abody(r, cr2):
                    plsc.addupdate(msum.at[pl.ds(r * 16, 16)],
                                   mbuf[pl.ds(r * 16, 16)])
                    return cr2
                lax.fori_loop(0, _NSEG // 16, abody, 0)
                return cr
            lax.fori_loop(1, _NSUB, tbody, 0)

            pltpu.sync_copy(msum, seg_a.at[pl.ds(base, _NSEG)])

    return segsum_a


def _make_segsum_b(n_half, ids_ofs):
    nfull = n_half // _C
    tail = n_half - nfull * _C
    seg_per = _NSEG // _NSUB
    mesh = plsc.VectorSubcoreMesh(core_axis_name="c", subcore_axis_name="s")

    @functools.partial(
        pl.kernel,
        mesh=mesh,
        out_type=(
            jax.ShapeDtypeStruct((_NSEG * 16,), jnp.float32),
            jax.ShapeDtypeStruct((_NSUB, _NSEG * 16), jnp.float32),
        ),
        scratch_types=[
            pltpu.VMEM((_C,), jnp.int32),
            pltpu.VMEM((_C * 16,), jnp.float32),
            pltpu.VMEM((_NSEG * 16,), jnp.float32),
            pltpu.VMEM((_NSEG,), jnp.float32),
            pltpu.VMEM((_NSEG,), jnp.float32),
        ],
    )
    def segsum_b(rows_hbm, ids_hbm, seg_a, out, slabs, ids_v, rows_v, acc,
                 mbuf, msum):
        cid = lax.axis_index("c")
        sid = lax.axis_index("s")

        @pl.when(cid == 0)
        def _():
            _sc_accum_phase(nfull, tail, ids_ofs, ids_hbm, rows_hbm, ids_v,
                            rows_v, acc, sid)
            pltpu.sync_copy(acc, slabs.at[sid])
            plsc.subcore_barrier()

            base = sid * (_NSEG // _NSUB) * 16
            pltpu.sync_copy(seg_a.at[pl.ds(base, _NSEG)], msum)

            def tbody(t, cr):
                pltpu.sync_copy(slabs.at[t, pl.ds(base, _NSEG)], mbuf)

                def abody(r, cr2):
                    plsc.addupdate(msum.at[pl.ds(r * 16, 16)],
                                   mbuf[pl.ds(r * 16, 16)])
                    return cr2
                lax.fori_loop(0, _NSEG // 16, abody, 0)
                return cr
            lax.fori_loop(0, _NSUB, tbody, 0)

            pltpu.sync_copy(msum, out.at[pl.ds(base, _NSEG)])

    return segsum_b


def kernel(pos, representation, vector_representation, W1v, W1a, b1a, W1b,
           b1b, W2v, W2a, b2a, W2b, b2b, batch):
    n = pos.shape[0]
    nh = _NHID
    n1 = n // 2
    v3 = jnp.transpose(vector_representation, (1, 0, 2))

    w2v3 = jnp.zeros((3 * nh, 6), jnp.float32)
    w2v3 = w2v3.at[0 * nh:1 * nh, 0].set(W2v[:, 0])
    w2v3 = w2v3.at[1 * nh:2 * nh, 1].set(W2v[:, 0])
    w2v3 = w2v3.at[2 * nh:3 * nh, 2].set(W2v[:, 0])
    w2v3 = w2v3.at[0 * nh:1 * nh, 3].set(W2v[:, 1])
    w2v3 = w2v3.at[1 * nh:2 * nh, 4].set(W2v[:, 1])
    w2v3 = w2v3.at[2 * nh:3 * nh, 5].set(W2v[:, 1])

    weights = (W1v, W1a[:_NIN], W1a[_NIN:], b1a.reshape(1, nh),
               W1b, b1b.reshape(1, 2 * nh), w2v3,
               W2a[:nh], W2a[nh:nh + 1], b2a.reshape(1, nh),
               W2b, b2b.reshape(1, 2))

    o1 = _atom_mlp(representation, v3, pos, *weights, n1, 0)
    o2 = _atom_mlp(representation, v3, pos, *weights, n1, n1 // _BLK)

    seg_a, _ = _make_segsum_a(n1)(o1.reshape(-1), batch)
    seg_flat, _ = _make_segsum_b(n1, n1)(o2.reshape(-1), batch, seg_a)
    seg = seg_flat.reshape(_NSEG, 16)
    y = seg[:, :3]
    y_vector = seg[:, 3:6][..., None]
    return (y, y_vector)
